# baseline (speedup 1.0000x reference)
import numpy as np
import jax
import jax.numpy as jnp
from jax import lax
from jax.experimental import pallas as pl
from jax.experimental.pallas import tpu as pltpu
from jax.experimental.pallas import tpu_sc as plsc

MIN_RES = 16
MAX_RES = 512
NUM_LOD = 16
HASH_BANDWIDTH = 19
FEAT_DIM = 2
TABLE_SIZE = 2 ** HASH_BANDWIDTH
_b = np.exp((np.log(MAX_RES) - np.log(MIN_RES)) / (NUM_LOD - 1))
LODS = [int(1 + np.floor(MIN_RES * _b ** l)) for l in range(NUM_LOD)]
SIZES = [min(r ** 3, TABLE_SIZE) for r in LODS]
DENSE = [r ** 3 <= TABLE_SIZE for r in LODS]
AL_SIZES = [((s + 127) // 128) * 128 for s in SIZES]
OFF_AL = []
_acc = 0
for _s in AL_SIZES:
    OFF_AL.append(_acc)
    _acc += _s
TOTAL_AL = _acc

P1 = np.int32(2654435761 - 2 ** 32)
P2 = np.int32(805459861)
MASK = np.int32(TABLE_SIZE - 1)

N_PTS = 262144
NW = 32
NS = 16
CHUNK = 512
NSTEP = CHUNK // 16
K = CHUNK // 128
NCHUNK = N_PTS // (NW * CHUNK)
RSPLIT = 2048


def _pairs_body(*refs):
    col_hs = refs[:2 * NUM_LOD]
    pairs_h = refs[2 * NUM_LOD]
    va_v, vb_v, st_v = refs[2 * NUM_LOD + 1:]
    cid = lax.axis_index("c")
    sid = lax.axis_index("s")
    iota = lax.iota(jnp.int32, 16)
    zeros_i = jnp.zeros((16,), jnp.int32)
    ones_i = zeros_i + 1
    cbase = cid * TOTAL_AL

    for l in range(NUM_LOD):
        al = AL_SIZES[l]
        off = OFF_AL[l]
        nch = (al + RSPLIT - 1) // RSPLIT
        nrounds = (nch + NS - 1) // NS
        last_a = al - RSPLIT
        ca_h = col_hs[2 * l]
        cb_h = col_hs[2 * l + 1]

        def round_body(t, carry, ca_h=ca_h, cb_h=cb_h, off=off, nch=nch,
                       last_a=last_a):
            j = t * NS + sid

            @pl.when(j < nch)
            def _():
                a = jnp.minimum(j * RSPLIT, last_a)
                pltpu.sync_copy(ca_h.at[pl.ds(a, RSPLIT)], va_v)
                pltpu.sync_copy(cb_h.at[pl.ds(a, RSPLIT)], vb_v)

                def int_step(s, c2):
                    p0 = s * 16
                    iv = iota + p0
                    plsc.store_scatter(
                        st_v, [iv, zeros_i], va_v[pl.ds(p0, 16)])
                    plsc.store_scatter(
                        st_v, [iv, ones_i], vb_v[pl.ds(p0, 16)])
                    return c2
                lax.fori_loop(0, RSPLIT // 16, int_step, 0)
                pltpu.sync_copy(st_v,
                                pairs_h.at[pl.ds(cbase + off + a, RSPLIT)])
            return carry

        lax.fori_loop(0, nrounds, round_body, 0)


def _body(xs_h, ys_h, zs_h, pairs_h, out_h,
          xs_v, ys_v, zs_v, fx_v, fy_v, fz_v, idx_v, rows_v, out_v, sem):
    cid = lax.axis_index("c")
    sid = lax.axis_index("s")
    wid = sid * 2 + cid
    iota = lax.iota(jnp.int32, 16)
    zeros_i = jnp.zeros((16,), jnp.int32)
    ones_i = zeros_i + 1
    cbase = cid * TOTAL_AL

    def chunk_body(ch, carry):
        base = wid * (NCHUNK * CHUNK) + ch * CHUNK
        pltpu.sync_copy(xs_h.at[pl.ds(base, CHUNK)], xs_v)
        pltpu.sync_copy(ys_h.at[pl.ds(base, CHUNK)], ys_v)
        pltpu.sync_copy(zs_h.at[pl.ds(base, CHUNK)], zs_v)

        for l in range(NUM_LOD):
            res = LODS[l]
            dense = DENSE[l]
            off = np.int32(OFF_AL[l])
            scale = np.float32(res - 1)
            cap = np.int32(res - 2)
            res2 = np.int32(res * res)
            resi = np.int32(res)

            def idx_step(s, c2, dense=dense, scale=scale, cap=cap,
                         res2=res2, resi=resi, off=off):
                p0 = s * 16
                x = xs_v[pl.ds(p0, 16)]
                y = ys_v[pl.ds(p0, 16)]
                z = zs_v[pl.ds(p0, 16)]
                sx = x * scale
                sy = y * scale
                sz = z * scale
                xi = jnp.minimum(sx.astype(jnp.int32), cap)
                yi = jnp.minimum(sy.astype(jnp.int32), cap)
                zi = jnp.minimum(sz.astype(jnp.int32), cap)
                fx_v[pl.ds(p0, 16)] = sx - xi.astype(jnp.float32)
                fy_v[pl.ds(p0, 16)] = sy - yi.astype(jnp.float32)
                fz_v[pl.ds(p0, 16)] = sz - zi.astype(jnp.float32)
                offv = off + cbase
                if dense:
                    ax = (xi + offv, xi + offv + 1)
                    ay = (yi * resi, yi * resi + resi)
                    az = (zi * res2, zi * res2 + res2)
                else:
                    ax = (xi, xi + 1)
                    ay = (yi * P1, yi * P1 + P1)
                    az = (zi * P2, zi * P2 + P2)
                t0 = s >> 3
                o = (s & 7) * 16
                c = 0
                for dx in (0, 1):
                    for dy in (0, 1):
                        for dz in (0, 1):
                            if dense:
                                idx = ax[dx] + ay[dy] + az[dz]
                            else:
                                idx = ((ax[dx] ^ ay[dy] ^ az[dz]) & MASK) + offv
                            idx_v[c * K + t0, pl.ds(o, 16)] = idx
                            c += 1
                return c2
            lax.fori_loop(0, NSTEP, idx_step, 0)

            def fire(t, c2):
                pltpu.make_async_copy(
                    pairs_h.at[idx_v.at[t]],
                    rows_v.at[pl.ds(t * 128, 128)],
                    sem).start()
                return c2
            lax.fori_loop(0, 8 * K, fire, 0)

            def drain(t, c2):
                pltpu.make_async_copy(
                    pairs_h.at[idx_v.at[t]],
                    rows_v.at[pl.ds(t * 128, 128)],
                    sem).wait()
                return c2
            lax.fori_loop(0, 8 * K, drain, 0)

            col0 = zeros_i + 2 * l
            col1 = col0 + 1

            def acc_step(s, c2, col0=col0, col1=col1):
                p0 = s * 16
                pvec = iota + p0
                fx = fx_v[pl.ds(p0, 16)]
                fy = fy_v[pl.ds(p0, 16)]
                fz = fz_v[pl.ds(p0, 16)]
                wx = (1.0 - fx, fx)
                wy = (1.0 - fy, fy)
                wz = (1.0 - fz, fz)
                acc0 = jnp.zeros((16,), jnp.float32)
                acc1 = jnp.zeros((16,), jnp.float32)
                c = 0
                for dx in (0, 1):
                    for dy in (0, 1):
                        for dz in (0, 1):
                            w = wx[dx] * wy[dy] * wz[dz]
                            rvec = pvec + c * CHUNK
                            g0 = plsc.load_gather(rows_v, [rvec, zeros_i])
                            g1 = plsc.load_gather(rows_v, [rvec, ones_i])
                            acc0 = acc0 + g0 * w
                            acc1 = acc1 + g1 * w
                            c += 1
                plsc.store_scatter(out_v, [pvec, col0], acc0)
                plsc.store_scatter(out_v, [pvec, col1], acc1)
                return c2
            lax.fori_loop(0, NSTEP, acc_step, 0)

        pltpu.sync_copy(out_v, out_h.at[pl.ds(base, CHUNK)])
        return carry

    lax.fori_loop(0, NCHUNK, chunk_body, 0)


_mesh = plsc.VectorSubcoreMesh(core_axis_name="c", subcore_axis_name="s")

_build_pairs = pl.kernel(
    _pairs_body,
    out_type=jax.ShapeDtypeStruct((2 * TOTAL_AL, FEAT_DIM), jnp.float32),
    mesh=_mesh,
    compiler_params=pltpu.CompilerParams(
        needs_layout_passes=False, use_tc_tiling_on_sc=False),
    scratch_types=[
        pltpu.VMEM((RSPLIT,), jnp.float32),
        pltpu.VMEM((RSPLIT,), jnp.float32),
        pltpu.VMEM((RSPLIT, FEAT_DIM), jnp.float32),
    ],
)

_hash_grid = pl.kernel(
    _body,
    out_type=jax.ShapeDtypeStruct((N_PTS, NUM_LOD * FEAT_DIM), jnp.float32),
    mesh=_mesh,
    compiler_params=pltpu.CompilerParams(
        needs_layout_passes=False, use_tc_tiling_on_sc=False),
    scratch_types=[
        pltpu.VMEM((CHUNK,), jnp.float32),
        pltpu.VMEM((CHUNK,), jnp.float32),
        pltpu.VMEM((CHUNK,), jnp.float32),
        pltpu.VMEM((CHUNK,), jnp.float32),
        pltpu.VMEM((CHUNK,), jnp.float32),
        pltpu.VMEM((CHUNK,), jnp.float32),
        pltpu.VMEM((8 * K, 128), jnp.int32),
        pltpu.VMEM((8 * CHUNK, FEAT_DIM), jnp.float32),
        pltpu.VMEM((CHUNK, NUM_LOD * FEAT_DIM), jnp.float32),
        pltpu.SemaphoreType.DMA,
    ],
)


def kernel(pts, grids):
    xs = pts[:, 0]
    ys = pts[:, 1]
    zs = pts[:, 2]
    cols = []
    for l, g in enumerate(grids):
        padn = AL_SIZES[l] - SIZES[l]
        cols.append(jnp.pad(g[:, 0], (0, padn)))
        cols.append(jnp.pad(g[:, 1], (0, padn)))
    pairs = _build_pairs(*cols)
    return _hash_grid(xs, ys, zs, pairs)


# speedup vs baseline: 1.0031x; 1.0031x over previous
import numpy as np
import jax
import jax.numpy as jnp
from jax import lax
from jax.experimental import pallas as pl
from jax.experimental.pallas import tpu as pltpu
from jax.experimental.pallas import tpu_sc as plsc

MIN_RES = 16
MAX_RES = 512
NUM_LOD = 16
HASH_BANDWIDTH = 19
FEAT_DIM = 2
TABLE_SIZE = 2 ** HASH_BANDWIDTH
_b = np.exp((np.log(MAX_RES) - np.log(MIN_RES)) / (NUM_LOD - 1))
LODS = [int(1 + np.floor(MIN_RES * _b ** l)) for l in range(NUM_LOD)]
SIZES = [min(r ** 3, TABLE_SIZE) for r in LODS]
DENSE = [r ** 3 <= TABLE_SIZE for r in LODS]
AL_SIZES = [((s + 127) // 128) * 128 for s in SIZES]
OFF_AL = []
_acc = 0
for _s in AL_SIZES:
    OFF_AL.append(_acc)
    _acc += _s
TOTAL_AL = _acc

P1 = np.int32(2654435761 - 2 ** 32)
P2 = np.int32(805459861)
MASK = np.int32(TABLE_SIZE - 1)

N_PTS = 262144
NW = 32
NS = 16
CHUNK = 512
NSTEP = CHUNK // 16
K = CHUNK // 128
NCHUNK = N_PTS // (NW * CHUNK)
RSPLIT = 2048


def _pairs_body(*refs):
    col_hs = refs[:2 * NUM_LOD]
    pairs_h = refs[2 * NUM_LOD]
    va_v, vb_v, st_v = refs[2 * NUM_LOD + 1:]
    cid = lax.axis_index("c")
    sid = lax.axis_index("s")
    iota = lax.iota(jnp.int32, 16)
    zeros_i = jnp.zeros((16,), jnp.int32)
    ones_i = zeros_i + 1
    cbase = cid * TOTAL_AL

    for l in range(NUM_LOD):
        al = AL_SIZES[l]
        off = OFF_AL[l]
        nch = (al + RSPLIT - 1) // RSPLIT
        nrounds = (nch + NS - 1) // NS
        last_a = al - RSPLIT
        ca_h = col_hs[2 * l]
        cb_h = col_hs[2 * l + 1]

        def round_body(t, carry, ca_h=ca_h, cb_h=cb_h, off=off, nch=nch,
                       last_a=last_a):
            j = t * NS + sid

            @pl.when(j < nch)
            def _():
                a = jnp.minimum(j * RSPLIT, last_a)
                pltpu.sync_copy(ca_h.at[pl.ds(a, RSPLIT)], va_v)
                pltpu.sync_copy(cb_h.at[pl.ds(a, RSPLIT)], vb_v)

                def int_step(s, c2):
                    p0 = s * 16
                    iv = iota + p0
                    plsc.store_scatter(
                        st_v, [iv, zeros_i], va_v[pl.ds(p0, 16)])
                    plsc.store_scatter(
                        st_v, [iv, ones_i], vb_v[pl.ds(p0, 16)])
                    return c2
                lax.fori_loop(0, RSPLIT // 16, int_step, 0)
                pltpu.sync_copy(st_v,
                                pairs_h.at[pl.ds(cbase + off + a, RSPLIT)])
            return carry

        lax.fori_loop(0, nrounds, round_body, 0)


def _body(xs_h, ys_h, zs_h, pairs_h, out_h,
          xs_v, ys_v, zs_v, fa_v, fb_v, idx0_v, idx1_v, rows0_v, rows1_v,
          out_v, sem0, sem1):
    cid = lax.axis_index("c")
    sid = lax.axis_index("s")
    wid = sid * 2 + cid
    iota = lax.iota(jnp.int32, 16)
    zeros_i = jnp.zeros((16,), jnp.int32)
    ones_i = zeros_i + 1
    cbase = cid * TOTAL_AL
    sems = (sem0, sem1)
    fbufs = (fa_v, fb_v)
    idxbufs = (idx0_v, idx1_v)
    rowbufs = (rows0_v, rows1_v)

    def stage(l, b):
        res = LODS[l]
        dense = DENSE[l]
        off = np.int32(OFF_AL[l])
        scale = np.float32(res - 1)
        cap = np.int32(res - 2)
        res2 = np.int32(res * res)
        resi = np.int32(res)

        def idx_step(s, c2):
            p0 = s * 16
            x = xs_v[pl.ds(p0, 16)]
            y = ys_v[pl.ds(p0, 16)]
            z = zs_v[pl.ds(p0, 16)]
            sx = x * scale
            sy = y * scale
            sz = z * scale
            xi = jnp.minimum(sx.astype(jnp.int32), cap)
            yi = jnp.minimum(sy.astype(jnp.int32), cap)
            zi = jnp.minimum(sz.astype(jnp.int32), cap)
            fbufs[b][0, pl.ds(p0, 16)] = sx - xi.astype(jnp.float32)
            fbufs[b][1, pl.ds(p0, 16)] = sy - yi.astype(jnp.float32)
            fbufs[b][2, pl.ds(p0, 16)] = sz - zi.astype(jnp.float32)
            offv = off + cbase
            if dense:
                ax = (xi + offv, xi + offv + 1)
                ay = (yi * resi, yi * resi + resi)
                az = (zi * res2, zi * res2 + res2)
            else:
                ax = (xi, xi + 1)
                ay = (yi * P1, yi * P1 + P1)
                az = (zi * P2, zi * P2 + P2)
            t0 = s >> 3
            o = (s & 7) * 16
            c = 0
            for dx in (0, 1):
                for dy in (0, 1):
                    for dz in (0, 1):
                        if dense:
                            idx = ax[dx] + ay[dy] + az[dz]
                        else:
                            idx = ((ax[dx] ^ ay[dy] ^ az[dz]) & MASK) + offv
                        idxbufs[b][c * K + t0, pl.ds(o, 16)] = idx
                        c += 1
            return c2
        lax.fori_loop(0, NSTEP, idx_step, 0)

        def fire(t, c2):
            pltpu.make_async_copy(
                pairs_h.at[idxbufs[b].at[t]],
                rowbufs[b].at[pl.ds(t * 128, 128)],
                sems[b]).start()
            return c2
        lax.fori_loop(0, 8 * K, fire, 0)

    def drain_acc(l, b):
        def drain(t, c2):
            pltpu.make_async_copy(
                pairs_h.at[idxbufs[b].at[t]],
                rowbufs[b].at[pl.ds(t * 128, 128)],
                sems[b]).wait()
            return c2
        lax.fori_loop(0, 8 * K, drain, 0)

        col0 = zeros_i + 2 * l
        col1 = col0 + 1
        rows_b = rowbufs[b]

        def acc_step(s, c2):
            p0 = s * 16
            pvec = iota + p0
            fx = fbufs[b][0, pl.ds(p0, 16)]
            fy = fbufs[b][1, pl.ds(p0, 16)]
            fz = fbufs[b][2, pl.ds(p0, 16)]
            wx = (1.0 - fx, fx)
            wy = (1.0 - fy, fy)
            wz = (1.0 - fz, fz)
            acc0 = jnp.zeros((16,), jnp.float32)
            acc1 = jnp.zeros((16,), jnp.float32)
            c = 0
            for dx in (0, 1):
                for dy in (0, 1):
                    for dz in (0, 1):
                        w = wx[dx] * wy[dy] * wz[dz]
                        rvec = pvec + c * CHUNK
                        g0 = plsc.load_gather(rows_b, [rvec, zeros_i])
                        g1 = plsc.load_gather(rows_b, [rvec, ones_i])
                        acc0 = acc0 + g0 * w
                        acc1 = acc1 + g1 * w
                        c += 1
            plsc.store_scatter(out_v, [pvec, col0], acc0)
            plsc.store_scatter(out_v, [pvec, col1], acc1)
            return c2
        lax.fori_loop(0, NSTEP, acc_step, 0)

    def chunk_body(ch, carry):
        base = wid * (NCHUNK * CHUNK) + ch * CHUNK
        pltpu.sync_copy(xs_h.at[pl.ds(base, CHUNK)], xs_v)
        pltpu.sync_copy(ys_h.at[pl.ds(base, CHUNK)], ys_v)
        pltpu.sync_copy(zs_h.at[pl.ds(base, CHUNK)], zs_v)

        for l in range(NUM_LOD):
            stage(l, l % 2)
            drain_acc(l, l % 2)

        pltpu.sync_copy(out_v, out_h.at[pl.ds(base, CHUNK)])
        return carry

    lax.fori_loop(0, NCHUNK, chunk_body, 0)


_mesh = plsc.VectorSubcoreMesh(core_axis_name="c", subcore_axis_name="s")

_build_pairs = pl.kernel(
    _pairs_body,
    out_type=jax.ShapeDtypeStruct((2 * TOTAL_AL, FEAT_DIM), jnp.float32),
    mesh=_mesh,
    compiler_params=pltpu.CompilerParams(
        needs_layout_passes=False, use_tc_tiling_on_sc=False),
    scratch_types=[
        pltpu.VMEM((RSPLIT,), jnp.float32),
        pltpu.VMEM((RSPLIT,), jnp.float32),
        pltpu.VMEM((RSPLIT, FEAT_DIM), jnp.float32),
    ],
)

_hash_grid = pl.kernel(
    _body,
    out_type=jax.ShapeDtypeStruct((N_PTS, NUM_LOD * FEAT_DIM), jnp.float32),
    mesh=_mesh,
    compiler_params=pltpu.CompilerParams(
        needs_layout_passes=False, use_tc_tiling_on_sc=False),
    scratch_types=[
        pltpu.VMEM((CHUNK,), jnp.float32),
        pltpu.VMEM((CHUNK,), jnp.float32),
        pltpu.VMEM((CHUNK,), jnp.float32),
        pltpu.VMEM((3, CHUNK), jnp.float32),
        pltpu.VMEM((3, CHUNK), jnp.float32),
        pltpu.VMEM((8 * K, 128), jnp.int32),
        pltpu.VMEM((8 * K, 128), jnp.int32),
        pltpu.VMEM((8 * CHUNK, FEAT_DIM), jnp.float32),
        pltpu.VMEM((8 * CHUNK, FEAT_DIM), jnp.float32),
        pltpu.VMEM((CHUNK, NUM_LOD * FEAT_DIM), jnp.float32),
        pltpu.SemaphoreType.DMA,
        pltpu.SemaphoreType.DMA,
    ],
)


def kernel(pts, grids):
    xs = pts[:, 0]
    ys = pts[:, 1]
    zs = pts[:, 2]
    cols = []
    for l, g in enumerate(grids):
        padn = AL_SIZES[l] - SIZES[l]
        cols.append(jnp.pad(g[:, 0], (0, padn)))
        cols.append(jnp.pad(g[:, 1], (0, padn)))
    pairs = _build_pairs(*cols)
    return _hash_grid(xs, ys, zs, pairs)
